# baseline (speedup 1.0000x reference)
import functools

import jax
import jax.numpy as jnp
from jax import lax
from jax.experimental import pallas as pl
from jax.experimental.pallas import tpu as pltpu
from jax.experimental.pallas import tpu_sc as plsc

B = 16384
EMB = 64
NC = 2
NS = 16
NW = NC * NS
BPW = B // NW
C = 128
NCHUNK = BPW // C
LANES = EMB // 16


def _transe_body(h_hbm, r_hbm, t_hbm, n_hbm, ent_hbm, rel_hbm,
                 score_hbm, neg_hbm,
                 hi, ri, ti, ni, hrows, rrows, trows, nrows, sem):
    wid = lax.axis_index("s") * NC + lax.axis_index("c")
    base = wid * BPW

    for chunk in range(NCHUNK):
        off = base + chunk * C
        pltpu.sync_copy(h_hbm.at[pl.ds(off, C)], hi)
        pltpu.sync_copy(r_hbm.at[pl.ds(off, C)], ri)
        pltpu.sync_copy(t_hbm.at[pl.ds(off, C)], ti)
        pltpu.sync_copy(n_hbm.at[pl.ds(off, C)], ni)

        g1 = pltpu.async_copy(ent_hbm.at[hi], hrows, sem)
        g2 = pltpu.async_copy(rel_hbm.at[ri], rrows, sem)
        g3 = pltpu.async_copy(ent_hbm.at[ti], trows, sem)
        g4 = pltpu.async_copy(ent_hbm.at[ni], nrows, sem)
        g1.wait()
        g2.wait()
        g3.wait()
        g4.wait()

        def row(i, _):
            for j in range(LANES):
                sl = pl.ds(j * 16, 16)
                hr = hrows[i, sl] + rrows[i, sl]
                trows[i, sl] = hr - trows[i, sl]
                nrows[i, sl] = hr - nrows[i, sl]
            return _

        lax.fori_loop(0, C, row, None)

        pltpu.sync_copy(trows, score_hbm.at[pl.ds(off, C)])
        pltpu.sync_copy(nrows, neg_hbm.at[pl.ds(off, C)])


@jax.jit
def _transe(h, r, t, n, ent, rel):
    mesh = plsc.VectorSubcoreMesh(core_axis_name="c", subcore_axis_name="s")
    f = functools.partial(
        pl.kernel,
        mesh=mesh,
        compiler_params=pltpu.CompilerParams(use_tc_tiling_on_sc=False),
        out_type=(
            jax.ShapeDtypeStruct((B, EMB), jnp.float32),
            jax.ShapeDtypeStruct((B, EMB), jnp.float32),
        ),
        scratch_types=[
            pltpu.VMEM((C,), jnp.int32),
            pltpu.VMEM((C,), jnp.int32),
            pltpu.VMEM((C,), jnp.int32),
            pltpu.VMEM((C,), jnp.int32),
            pltpu.VMEM((C, EMB), jnp.float32),
            pltpu.VMEM((C, EMB), jnp.float32),
            pltpu.VMEM((C, EMB), jnp.float32),
            pltpu.VMEM((C, EMB), jnp.float32),
            pltpu.SemaphoreType.DMA,
        ],
    )(_transe_body)
    return f(h, r, t, n, ent, rel)


def kernel(h, r, t, neg_t_idx, entity_emb, relation_emb):
    score, neg = _transe(
        h.astype(jnp.int32),
        r.astype(jnp.int32),
        t.astype(jnp.int32),
        neg_t_idx.astype(jnp.int32),
        entity_emb,
        relation_emb,
    )
    return score[:, None, :], neg[:, None, :]
